# speedup vs baseline: 1.1009x; 1.1009x over previous
import functools

import jax
import jax.numpy as jnp
from jax.experimental import pallas as pl
from jax.experimental.pallas import tpu as pltpu

N = 10000
E = 160000
D = 256
H = 4
C = 64
HC = H * C
NEG_SLOPE = 0.2
EPS = 1e-5

NPAD = 10240


def _matmul_body(x_ref, wl_ref, wr_ref, xl_ref, xr_ref):
    x = x_ref[...]
    xl_ref[...] = jnp.dot(x, wl_ref[...], preferred_element_type=jnp.float32)
    xr_ref[...] = jnp.dot(x, wr_ref[...], preferred_element_type=jnp.float32)


def _matmuls(x_pad, W_l, W_r):
    blk = 1024
    grid = NPAD // blk
    return pl.pallas_call(
        _matmul_body,
        grid=(grid,),
        in_specs=[
            pl.BlockSpec((blk, D), lambda i: (i, 0)),
            pl.BlockSpec((D, HC), lambda i: (0, 0)),
            pl.BlockSpec((D, HC), lambda i: (0, 0)),
        ],
        out_specs=[
            pl.BlockSpec((blk, HC), lambda i: (i, 0)),
            pl.BlockSpec((blk, HC), lambda i: (i, 0)),
        ],
        out_shape=[
            jax.ShapeDtypeStruct((NPAD, HC), jnp.float32),
            jax.ShapeDtypeStruct((NPAD, HC), jnp.float32),
        ],
    )(x_pad, W_l, W_r)


def kernel(x, edge_index, W_l, W_r, att, bias, gn_weight, gn_bias, gn_mean_scale):
    x_pad = jnp.zeros((NPAD, D), jnp.float32).at[:N].set(x)
    xl_pad, xr_pad = _matmuls(x_pad, W_l, W_r)
    x_l = xl_pad[:N].reshape(N, H, C)
    x_r = xr_pad[:N].reshape(N, H, C)

    loop = jnp.arange(N, dtype=edge_index.dtype)
    src = jnp.concatenate([edge_index[0], loop])
    dst = jnp.concatenate([edge_index[1], loop])

    e = jax.nn.leaky_relu(x_l[src] + x_r[dst], negative_slope=NEG_SLOPE)
    logits = jnp.sum(e * att[None, :, :], axis=-1)
    p = jnp.exp(logits)
    denom = jax.ops.segment_sum(p, dst, num_segments=N)
    msg = x_l[src] * p[:, :, None]
    acc = jax.ops.segment_sum(msg, dst, num_segments=N)
    out = (acc / denom[:, :, None]).reshape(N, HC) + bias

    mean = jnp.mean(out, axis=0)
    centered = out - gn_mean_scale * mean
    var = jnp.mean(centered * centered, axis=0)
    return gn_weight * centered / jnp.sqrt(var + EPS) + gn_bias
